# baseline (speedup 1.0000x reference)
import functools

import jax
import jax.numpy as jnp
import numpy as np
from jax import lax
from jax.experimental import pallas as pl
from jax.experimental.pallas import tpu as pltpu
from jax.experimental.pallas import tpu_sc as plsc

BEV_X_RANGE = (-40.0, 40.0)
BEV_Y_RANGE = (0.0, 80.0)
BETA = 1.0
NPX = 10

NC, NS, L = 2, 16, 16
NW = NC * NS


def _make_kernel(B, D, H, W):
    ndet = B * D
    dets_per_w = ndet // NW
    pts_per_det = NPX * NPX
    pts_per_w = dets_per_w * pts_per_det
    nvreg = pts_per_w // L
    nrows = dets_per_w * NPX
    npts = ndet * pts_per_det

    mesh = plsc.VectorSubcoreMesh(core_axis_name="c", subcore_axis_name="s")

    @functools.partial(
        pl.kernel,
        out_type=(
            jax.ShapeDtypeStruct((npts * 5,), jnp.float32),
            jax.ShapeDtypeStruct((npts,), jnp.float32),
        ),
        mesh=mesh,
        compiler_params=pltpu.CompilerParams(needs_layout_passes=False),
        scratch_types=[
            pltpu.VMEM((dets_per_w * 4,), jnp.float32),
            pltpu.VMEM((nrows,), jnp.int32),
            pltpu.VMEM((nrows,), jnp.float32),
            pltpu.VMEM((pts_per_w,), jnp.int32),
            pltpu.VMEM((pts_per_w,), jnp.int32),
            pltpu.VMEM((pts_per_w,), jnp.float32),
            pltpu.VMEM((pts_per_w,), jnp.float32),
            pltpu.VMEM((4 * L,), jnp.float32),
            pltpu.VMEM((nrows,), jnp.int32),
            pltpu.VMEM((nrows, W), jnp.float32),
            pltpu.VMEM((nrows, W), jnp.float32),
            pltpu.VMEM((pts_per_w * 5,), jnp.float32),
            pltpu.VMEM((pts_per_w,), jnp.float32),
            pltpu.SemaphoreType.DMA,
            pltpu.SemaphoreType.DMA,
        ],
    )
    def k(depth_hbm, lv_hbm, bboxf_hbm, rk_hbm, rt_hbm, kk_hbm, row_hbm,
          ti_hbm, tj_hbm, params_hbm,
          pts_hbm, conf_hbm,
          bboxf_v, rk_v, rt_v, kk_v, row_v, ti_v, tj_v, params_v,
          rowidx_v, drows_v, lrows_v, pts_v, conf_v,
          sem1, sem2):
        wid = lax.axis_index("s") * NC + lax.axis_index("c")
        det0 = wid * dets_per_w
        b = wid // (NW // B)

        pltpu.sync_copy(bboxf_hbm.at[pl.ds(det0 * 4, dets_per_w * 4)], bboxf_v)
        pltpu.sync_copy(rk_hbm, rk_v)
        pltpu.sync_copy(rt_hbm, rt_v)
        pltpu.sync_copy(kk_hbm, kk_v)
        pltpu.sync_copy(row_hbm, row_v)
        pltpu.sync_copy(ti_hbm, ti_v)
        pltpu.sync_copy(tj_hbm, tj_v)
        pltpu.sync_copy(params_hbm, params_v)

        iota = lax.broadcasted_iota(jnp.int32, (L,), 0)
        wmax = jnp.float32(W - 1)
        hmax = jnp.float32(H - 1)

        for n in range(nrows // L):
            rk = rk_v[pl.ds(n * L, L)]
            rt = rt_v[pl.ds(n * L, L)]
            rk4 = rk * 4
            y1 = plsc.load_gather(bboxf_v, [rk4 + 1])
            y2 = plsc.load_gather(bboxf_v, [rk4 + 3])
            y1c = jnp.clip(y1, 0.0, hmax)
            y2c = jnp.clip(y2, 0.0, hmax)
            v = y1c + rt * (y2c - y1c)
            vi = jnp.clip(v.astype(jnp.int32), 0, H - 1)
            rowidx_v[pl.ds(n * L, L)] = b * H + vi

        cp1 = pltpu.async_copy(depth_hbm.at[rowidx_v], drows_v, sem1)
        cp2 = pltpu.async_copy(lv_hbm.at[rowidx_v], lrows_v, sem2)
        cp1.wait()
        cp2.wait()

        fxv = params_v[pl.ds(0 * L, L)]
        fyv = params_v[pl.ds(1 * L, L)]
        cxv = params_v[pl.ds(2 * L, L)]
        cyv = params_v[pl.ds(3 * L, L)]

        def body(n, carry):
            base = n * L
            kk4 = kk_v[pl.ds(base, L)] * 4
            row = row_v[pl.ds(base, L)]
            ti = ti_v[pl.ds(base, L)]
            tj = tj_v[pl.ds(base, L)]
            x1 = plsc.load_gather(bboxf_v, [kk4])
            y1 = plsc.load_gather(bboxf_v, [kk4 + 1])
            x2 = plsc.load_gather(bboxf_v, [kk4 + 2])
            y2 = plsc.load_gather(bboxf_v, [kk4 + 3])
            x1c = jnp.clip(x1, 0.0, wmax)
            x2c = jnp.clip(x2, 0.0, wmax)
            y1c = jnp.clip(y1, 0.0, hmax)
            y2c = jnp.clip(y2, 0.0, hmax)
            u = x1c + tj * (x2c - x1c)
            v = y1c + ti * (y2c - y1c)
            ui = jnp.clip(u.astype(jnp.int32), 0, W - 1)
            dep = plsc.load_gather(drows_v, [row, ui])
            lv = plsc.load_gather(lrows_v, [row, ui])
            conf = jnp.clip(jnp.exp(-BETA * lv), 0.0, 1.0)
            x_cam = (u - cxv) * dep / fxv
            y_cam = (v - cyv) * dep / fyv
            x_r = dep
            y_r = -x_cam
            z_r = -y_cam
            mask = ((dep > 0.5)
                    & (x_r > BEV_Y_RANGE[0]) & (x_r < BEV_Y_RANGE[1])
                    & (y_r > BEV_X_RANGE[0]) & (y_r < BEV_X_RANGE[1]))
            mf = jnp.where(mask, jnp.float32(1.0), jnp.float32(0.0))
            l5 = (iota + base) * 5
            plsc.store_scatter(pts_v, [l5], x_r * mf)
            plsc.store_scatter(pts_v, [l5 + 1], y_r * mf)
            plsc.store_scatter(pts_v, [l5 + 2], z_r * mf)
            plsc.store_scatter(pts_v, [l5 + 3], jnp.zeros((L,), jnp.float32))
            plsc.store_scatter(pts_v, [l5 + 4], jnp.float32(10.0) * mf)
            conf_v[pl.ds(base, L)] = conf * mf
            return carry

        lax.fori_loop(0, nvreg, body, 0)

        pltpu.sync_copy(pts_v, pts_hbm.at[pl.ds(wid * pts_per_w * 5, pts_per_w * 5)])
        pltpu.sync_copy(conf_v, conf_hbm.at[pl.ds(wid * pts_per_w, pts_per_w)])

    return k


def kernel(images, depth_map, log_var_map, bboxes, intrinsic):
    del images
    B, _, H, W = depth_map.shape
    D = bboxes.shape[1]
    ndet = B * D
    pts_per_w = (ndet // NW) * NPX * NPX
    nrows = (ndet // NW) * NPX

    depth_rows = depth_map.reshape(B * H, W)
    lv_rows = log_var_map.reshape(B * H, W)
    bboxf = bboxes.reshape(ndet * 4)
    t = jnp.linspace(0.0, 1.0, NPX)

    lr = np.arange(nrows)
    rk = jnp.asarray(lr // NPX, jnp.int32)
    rt = t[lr % NPX]
    lp = np.arange(pts_per_w)
    kk = jnp.asarray(lp // (NPX * NPX), jnp.int32)
    row = jnp.asarray((lp // (NPX * NPX)) * NPX + (lp // NPX) % NPX, jnp.int32)
    ti = t[(lp // NPX) % NPX]
    tj = t[lp % NPX]

    params = jnp.concatenate([
        jnp.broadcast_to(intrinsic[0, 0], (L,)),
        jnp.broadcast_to(intrinsic[1, 1], (L,)),
        jnp.broadcast_to(intrinsic[0, 2], (L,)),
        jnp.broadcast_to(intrinsic[1, 2], (L,)),
    ])

    k = _make_kernel(B, D, H, W)
    pts_flat, conf = k(depth_rows, lv_rows, bboxf, rk, rt, kk, row, ti, tj,
                       params)
    return pts_flat.reshape(ndet * NPX * NPX, 5), conf


# speedup vs baseline: 1.8023x; 1.8023x over previous
import functools

import jax
import jax.numpy as jnp
import numpy as np
from jax import lax
from jax.experimental import pallas as pl
from jax.experimental.pallas import tpu as pltpu
from jax.experimental.pallas import tpu_sc as plsc

BEV_X_RANGE = (-40.0, 40.0)
BEV_Y_RANGE = (0.0, 80.0)
BETA = 1.0
NPX = 10

NC, NS, L = 2, 16, 16
NW = NC * NS


def _make_kernel(B, D, H, W):
    ndet = B * D
    dets_per_w = ndet // NW
    pts_per_det = NPX * NPX
    pts_per_w = dets_per_w * pts_per_det
    nvreg = pts_per_w // L
    nrows = dets_per_w * NPX
    npts = ndet * pts_per_det

    mesh = plsc.VectorSubcoreMesh(core_axis_name="c", subcore_axis_name="s")

    @functools.partial(
        pl.kernel,
        out_type=(
            jax.ShapeDtypeStruct((5 * npts,), jnp.float32),
            jax.ShapeDtypeStruct((npts,), jnp.float32),
        ),
        mesh=mesh,
        compiler_params=pltpu.CompilerParams(needs_layout_passes=False),
        scratch_types=[
            pltpu.VMEM((dets_per_w * 4,), jnp.float32),
            pltpu.VMEM((16,), jnp.float32),
            pltpu.VMEM((4 * 16,), jnp.int32),
            pltpu.VMEM((nrows,), jnp.int32),
            pltpu.VMEM((nrows,), jnp.float32),
            pltpu.VMEM((pts_per_w,), jnp.int32),
            pltpu.VMEM((pts_per_w,), jnp.int32),
            pltpu.VMEM((pts_per_w,), jnp.float32),
            pltpu.VMEM((pts_per_w,), jnp.float32),
            pltpu.VMEM((nrows,), jnp.int32),
            pltpu.VMEM((nrows, W), jnp.float32),
            pltpu.VMEM((nrows, W), jnp.float32),
            pltpu.VMEM((pts_per_w,), jnp.float32),
            pltpu.VMEM((pts_per_w,), jnp.float32),
            pltpu.VMEM((pts_per_w,), jnp.float32),
            pltpu.VMEM((pts_per_w,), jnp.float32),
            pltpu.VMEM((pts_per_w,), jnp.float32),
            pltpu.VMEM((pts_per_w,), jnp.float32),
            pltpu.SemaphoreType.DMA,
            pltpu.SemaphoreType.DMA,
        ],
    )
    def k(depth_hbm, lv_hbm, bboxf_hbm, intr_hbm, pidx_hbm, rk4_hbm, rt_hbm,
          kk4_hbm, row_hbm, ti_hbm, tj_hbm,
          pts_hbm, conf_hbm,
          bboxf_v, intr_v, pidx_v, rk4_v, rt_v, kk4_v, row_v, ti_v, tj_v,
          rowidx_v, drows_v, lrows_v, p0_v, p1_v, p2_v, p3_v, p4_v, conf_v,
          sem1, sem2):
        wid = lax.axis_index("s") * NC + lax.axis_index("c")
        det0 = wid * dets_per_w
        b = wid // (NW // B)

        pltpu.sync_copy(bboxf_hbm.at[pl.ds(det0 * 4, dets_per_w * 4)], bboxf_v)
        pltpu.sync_copy(intr_hbm, intr_v)
        pltpu.sync_copy(pidx_hbm, pidx_v)
        pltpu.sync_copy(rk4_hbm, rk4_v)
        pltpu.sync_copy(rt_hbm, rt_v)
        pltpu.sync_copy(kk4_hbm, kk4_v)
        pltpu.sync_copy(row_hbm, row_v)
        pltpu.sync_copy(ti_hbm, ti_v)
        pltpu.sync_copy(tj_hbm, tj_v)

        wmax = jnp.float32(W - 1)
        hmax = jnp.float32(H - 1)

        for n in range(nrows // L):
            rk4 = rk4_v[pl.ds(n * L, L)]
            rt = rt_v[pl.ds(n * L, L)]
            y1 = plsc.load_gather(bboxf_v, [rk4 + 1])
            y2 = plsc.load_gather(bboxf_v, [rk4 + 3])
            y1c = jnp.clip(y1, 0.0, hmax)
            y2c = jnp.clip(y2, 0.0, hmax)
            v = y1c + rt * (y2c - y1c)
            vi = jnp.clip(v.astype(jnp.int32), 0, H - 1)
            rowidx_v[pl.ds(n * L, L)] = b * H + vi

        cp1 = pltpu.async_copy(depth_hbm.at[rowidx_v], drows_v, sem1)
        cp2 = pltpu.async_copy(lv_hbm.at[rowidx_v], lrows_v, sem2)

        fxv = plsc.load_gather(intr_v, [pidx_v[pl.ds(0 * L, L)]])
        fyv = plsc.load_gather(intr_v, [pidx_v[pl.ds(1 * L, L)]])
        cxv = plsc.load_gather(intr_v, [pidx_v[pl.ds(2 * L, L)]])
        cyv = plsc.load_gather(intr_v, [pidx_v[pl.ds(3 * L, L)]])
        rfxv = jnp.float32(1.0) / fxv
        rfyv = jnp.float32(1.0) / fyv

        cp1.wait()
        cp2.wait()

        for n in range(nvreg):
            base = n * L
            kk4 = kk4_v[pl.ds(base, L)]
            row = row_v[pl.ds(base, L)]
            ti = ti_v[pl.ds(base, L)]
            tj = tj_v[pl.ds(base, L)]
            x1 = plsc.load_gather(bboxf_v, [kk4])
            y1 = plsc.load_gather(bboxf_v, [kk4 + 1])
            x2 = plsc.load_gather(bboxf_v, [kk4 + 2])
            y2 = plsc.load_gather(bboxf_v, [kk4 + 3])
            x1c = jnp.clip(x1, 0.0, wmax)
            x2c = jnp.clip(x2, 0.0, wmax)
            y1c = jnp.clip(y1, 0.0, hmax)
            y2c = jnp.clip(y2, 0.0, hmax)
            u = x1c + tj * (x2c - x1c)
            v = y1c + ti * (y2c - y1c)
            ui = jnp.clip(u.astype(jnp.int32), 0, W - 1)
            dep = plsc.load_gather(drows_v, [row, ui])
            lv = plsc.load_gather(lrows_v, [row, ui])
            conf = jnp.clip(jnp.exp(-BETA * lv), 0.0, 1.0)
            x_cam = (u - cxv) * dep * rfxv
            y_cam = (v - cyv) * dep * rfyv
            x_r = dep
            y_r = -x_cam
            z_r = -y_cam
            mask = ((dep > 0.5)
                    & (x_r > BEV_Y_RANGE[0]) & (x_r < BEV_Y_RANGE[1])
                    & (y_r > BEV_X_RANGE[0]) & (y_r < BEV_X_RANGE[1]))
            mf = jnp.where(mask, jnp.float32(1.0), jnp.float32(0.0))
            sl = pl.ds(base, L)
            p0_v[sl] = x_r * mf
            p1_v[sl] = y_r * mf
            p2_v[sl] = z_r * mf
            p3_v[sl] = jnp.zeros((L,), jnp.float32)
            p4_v[sl] = jnp.float32(10.0) * mf
            conf_v[sl] = conf * mf

        base_out = wid * pts_per_w
        pltpu.sync_copy(p0_v, pts_hbm.at[pl.ds(base_out, pts_per_w)])
        pltpu.sync_copy(p1_v, pts_hbm.at[pl.ds(npts + base_out, pts_per_w)])
        pltpu.sync_copy(p2_v, pts_hbm.at[pl.ds(2 * npts + base_out, pts_per_w)])
        pltpu.sync_copy(p3_v, pts_hbm.at[pl.ds(3 * npts + base_out, pts_per_w)])
        pltpu.sync_copy(p4_v, pts_hbm.at[pl.ds(4 * npts + base_out, pts_per_w)])
        pltpu.sync_copy(conf_v, conf_hbm.at[pl.ds(base_out, pts_per_w)])

    return k


def kernel(images, depth_map, log_var_map, bboxes, intrinsic):
    del images
    B, _, H, W = depth_map.shape
    D = bboxes.shape[1]
    ndet = B * D
    pts_per_w = (ndet // NW) * NPX * NPX
    nrows = (ndet // NW) * NPX

    depth_rows = depth_map.reshape(B * H, W)
    lv_rows = log_var_map.reshape(B * H, W)
    bboxf = bboxes.reshape(ndet * 4)
    intr_flat = jnp.concatenate(
        [intrinsic.reshape(9), jnp.zeros((7,), jnp.float32)])

    t = np.linspace(0.0, 1.0, NPX).astype(np.float32)
    lr = np.arange(nrows)
    rk4 = jnp.asarray((lr // NPX) * 4, jnp.int32)
    rt = jnp.asarray(t[lr % NPX])
    lp = np.arange(pts_per_w)
    kk4 = jnp.asarray((lp // (NPX * NPX)) * 4, jnp.int32)
    row = jnp.asarray((lp // (NPX * NPX)) * NPX + (lp // NPX) % NPX, jnp.int32)
    ti = jnp.asarray(t[(lp // NPX) % NPX])
    tj = jnp.asarray(t[lp % NPX])

    pidx = jnp.asarray(np.repeat(np.array([0, 4, 2, 5]), 16), jnp.int32)

    k = _make_kernel(B, D, H, W)
    pts5, conf = k(depth_rows, lv_rows, bboxf, intr_flat, pidx, rk4, rt, kk4,
                   row, ti, tj)
    return pts5.reshape(5, ndet * NPX * NPX).T, conf


# speedup vs baseline: 2.2853x; 1.2680x over previous
import functools

import jax
import jax.numpy as jnp
import numpy as np
from jax import lax
from jax.experimental import pallas as pl
from jax.experimental.pallas import tpu as pltpu
from jax.experimental.pallas import tpu_sc as plsc

BEV_X_RANGE = (-40.0, 40.0)
BEV_Y_RANGE = (0.0, 80.0)
BETA = 1.0
NPX = 10

NC, NS, L = 2, 16, 16
NW = NC * NS


def _make_kernel(B, D, H, W):
    ndet = B * D
    dets_per_w = ndet // NW
    pts_per_det = NPX * NPX
    pts_per_w = dets_per_w * pts_per_det
    nvreg = pts_per_w // L
    nrows = dets_per_w * NPX
    npts = ndet * pts_per_det

    mesh = plsc.VectorSubcoreMesh(core_axis_name="c", subcore_axis_name="s")

    PIDX = 0
    RK4 = 64
    RT = RK4 + nrows
    KK4 = RT + nrows
    ROW = KK4 + pts_per_w
    TI = ROW + pts_per_w
    TJ = TI + pts_per_w

    @functools.partial(
        pl.kernel,
        out_type=(
            jax.ShapeDtypeStruct((5 * npts,), jnp.float32),
            jax.ShapeDtypeStruct((npts,), jnp.float32),
        ),
        mesh=mesh,
        compiler_params=pltpu.CompilerParams(needs_layout_passes=False),
        scratch_types=[
            pltpu.VMEM((dets_per_w * 4,), jnp.float32),
            pltpu.VMEM((16,), jnp.float32),
            pltpu.VMEM((64 + nrows * 2 + pts_per_w * 4,), jnp.int32),
            pltpu.VMEM((nrows,), jnp.int32),
            pltpu.VMEM((nrows, W), jnp.float32),
            pltpu.VMEM((nrows, W), jnp.float32),
            pltpu.VMEM((pts_per_w,), jnp.float32),
            pltpu.VMEM((pts_per_w,), jnp.float32),
            pltpu.VMEM((pts_per_w,), jnp.float32),
            pltpu.VMEM((pts_per_w,), jnp.float32),
            pltpu.VMEM((pts_per_w,), jnp.float32),
            pltpu.VMEM((pts_per_w,), jnp.float32),
            pltpu.SemaphoreType.DMA,
            pltpu.SemaphoreType.DMA,
        ],
    )
    def k(bbi_hbm, depth_hbm, lv_hbm, lut_hbm,
          pts_hbm, conf_hbm,
          bboxf_v, intr_v, lut_v,
          rowidx_v, drows_v, lrows_v, p0_v, p1_v, p2_v, p3_v, p4_v, conf_v,
          sem1, sem2):
        wid = lax.axis_index("s") * NC + lax.axis_index("c")
        det0 = wid * dets_per_w
        b = wid // (NW // B)

        pltpu.sync_copy(bbi_hbm.at[pl.ds(det0 * 4, dets_per_w * 4)], bboxf_v)
        pltpu.sync_copy(bbi_hbm.at[pl.ds(ndet * 4, 16)], intr_v)
        pltpu.sync_copy(lut_hbm, lut_v)

        wmax = jnp.float32(W - 1)
        hmax = jnp.float32(H - 1)

        for n in range(nrows // L):
            rk4 = lut_v[pl.ds(RK4 + n * L, L)]
            rt = plsc.bitcast(lut_v[pl.ds(RT + n * L, L)], jnp.float32)
            y1 = plsc.load_gather(bboxf_v, [rk4 + 1])
            y2 = plsc.load_gather(bboxf_v, [rk4 + 3])
            y1c = jnp.clip(y1, 0.0, hmax)
            y2c = jnp.clip(y2, 0.0, hmax)
            v = y1c + rt * (y2c - y1c)
            vi = jnp.clip(v.astype(jnp.int32), 0, H - 1)
            rowidx_v[pl.ds(n * L, L)] = b * H + vi

        cp1 = pltpu.async_copy(depth_hbm.at[rowidx_v], drows_v, sem1)
        cp2 = pltpu.async_copy(lv_hbm.at[rowidx_v], lrows_v, sem2)

        fxv = plsc.load_gather(intr_v, [lut_v[pl.ds(PIDX + 0 * L, L)]])
        fyv = plsc.load_gather(intr_v, [lut_v[pl.ds(PIDX + 1 * L, L)]])
        cxv = plsc.load_gather(intr_v, [lut_v[pl.ds(PIDX + 2 * L, L)]])
        cyv = plsc.load_gather(intr_v, [lut_v[pl.ds(PIDX + 3 * L, L)]])
        rfxv = jnp.float32(1.0) / fxv
        rfyv = jnp.float32(1.0) / fyv

        cp1.wait()
        cp2.wait()

        for n in range(nvreg):
            base = n * L
            kk4 = lut_v[pl.ds(KK4 + base, L)]
            row = lut_v[pl.ds(ROW + base, L)]
            ti = plsc.bitcast(lut_v[pl.ds(TI + base, L)], jnp.float32)
            tj = plsc.bitcast(lut_v[pl.ds(TJ + base, L)], jnp.float32)
            x1 = plsc.load_gather(bboxf_v, [kk4])
            y1 = plsc.load_gather(bboxf_v, [kk4 + 1])
            x2 = plsc.load_gather(bboxf_v, [kk4 + 2])
            y2 = plsc.load_gather(bboxf_v, [kk4 + 3])
            x1c = jnp.clip(x1, 0.0, wmax)
            x2c = jnp.clip(x2, 0.0, wmax)
            y1c = jnp.clip(y1, 0.0, hmax)
            y2c = jnp.clip(y2, 0.0, hmax)
            u = x1c + tj * (x2c - x1c)
            v = y1c + ti * (y2c - y1c)
            ui = jnp.clip(u.astype(jnp.int32), 0, W - 1)
            dep = plsc.load_gather(drows_v, [row, ui])
            lv = plsc.load_gather(lrows_v, [row, ui])
            conf = jnp.clip(jnp.exp(-BETA * lv), 0.0, 1.0)
            x_cam = (u - cxv) * dep * rfxv
            y_cam = (v - cyv) * dep * rfyv
            x_r = dep
            y_r = -x_cam
            z_r = -y_cam
            mask = ((dep > 0.5)
                    & (x_r > BEV_Y_RANGE[0]) & (x_r < BEV_Y_RANGE[1])
                    & (y_r > BEV_X_RANGE[0]) & (y_r < BEV_X_RANGE[1]))
            mf = jnp.where(mask, jnp.float32(1.0), jnp.float32(0.0))
            sl = pl.ds(base, L)
            p0_v[sl] = x_r * mf
            p1_v[sl] = y_r * mf
            p2_v[sl] = z_r * mf
            p3_v[sl] = jnp.zeros((L,), jnp.float32)
            p4_v[sl] = jnp.float32(10.0) * mf
            conf_v[sl] = conf * mf

        base_out = wid * pts_per_w
        pltpu.sync_copy(p0_v, pts_hbm.at[pl.ds(base_out, pts_per_w)])
        pltpu.sync_copy(p1_v, pts_hbm.at[pl.ds(npts + base_out, pts_per_w)])
        pltpu.sync_copy(p2_v, pts_hbm.at[pl.ds(2 * npts + base_out, pts_per_w)])
        pltpu.sync_copy(p3_v, pts_hbm.at[pl.ds(3 * npts + base_out, pts_per_w)])
        pltpu.sync_copy(p4_v, pts_hbm.at[pl.ds(4 * npts + base_out, pts_per_w)])
        pltpu.sync_copy(conf_v, conf_hbm.at[pl.ds(base_out, pts_per_w)])

    return k


def kernel(images, depth_map, log_var_map, bboxes, intrinsic):
    del images
    B, _, H, W = depth_map.shape
    D = bboxes.shape[1]
    ndet = B * D
    pts_per_w = (ndet // NW) * NPX * NPX
    nrows = (ndet // NW) * NPX

    depth_rows = depth_map.reshape(B * H, W)
    lv_rows = log_var_map.reshape(B * H, W)
    bbi = jnp.concatenate([
        bboxes.reshape(ndet * 4),
        intrinsic.reshape(9),
        jnp.zeros((7,), jnp.float32),
    ])

    t = np.linspace(0.0, 1.0, NPX).astype(np.float32)
    lr = np.arange(nrows)
    lp = np.arange(pts_per_w)
    lut = np.concatenate([
        np.repeat(np.array([0, 4, 2, 5]), 16).astype(np.int32),
        ((lr // NPX) * 4).astype(np.int32),
        t[lr % NPX].view(np.int32),
        ((lp // (NPX * NPX)) * 4).astype(np.int32),
        ((lp // (NPX * NPX)) * NPX + (lp // NPX) % NPX).astype(np.int32),
        t[(lp // NPX) % NPX].view(np.int32),
        t[lp % NPX].view(np.int32),
    ])
    lut = jnp.asarray(lut)

    k = _make_kernel(B, D, H, W)
    pts5, conf = k(bbi, depth_rows, lv_rows, lut)
    return pts5.reshape(5, ndet * NPX * NPX).T, conf


# speedup vs baseline: 2.3768x; 1.0400x over previous
import functools

import jax
import jax.numpy as jnp
import numpy as np
from jax import lax
from jax.experimental import pallas as pl
from jax.experimental.pallas import tpu as pltpu
from jax.experimental.pallas import tpu_sc as plsc

BEV_X_RANGE = (-40.0, 40.0)
BEV_Y_RANGE = (0.0, 80.0)
BETA = 1.0
NPX = 10

NC, NS, L = 2, 16, 16
NW = NC * NS


def _make_kernel(B, D, H, W):
    ndet = B * D
    dets_per_w = ndet // NW
    pts_per_det = NPX * NPX
    pts_per_w = dets_per_w * pts_per_det
    nvreg = pts_per_w // L
    nrows = dets_per_w * NPX
    npts = ndet * pts_per_det
    half_rows = nrows // 2
    half_vreg = nvreg // 2

    mesh = plsc.VectorSubcoreMesh(core_axis_name="c", subcore_axis_name="s")

    PIDX = 0
    XL = PIDX + 64
    RK4 = XL + 16
    RT = RK4 + nrows
    KK = RT + nrows
    ROW = KK + pts_per_w
    TJ = ROW + pts_per_w
    LUT_LEN = TJ + pts_per_w

    @functools.partial(
        pl.kernel,
        out_type=(
            jax.ShapeDtypeStruct((5 * npts,), jnp.float32),
            jax.ShapeDtypeStruct((npts,), jnp.float32),
        ),
        mesh=mesh,
        compiler_params=pltpu.CompilerParams(needs_layout_passes=False),
        scratch_types=[
            pltpu.VMEM((dets_per_w * 4,), jnp.float32),
            pltpu.VMEM((16,), jnp.float32),
            pltpu.VMEM((LUT_LEN,), jnp.int32),
            pltpu.VMEM((nrows,), jnp.int32),
            pltpu.VMEM((nrows,), jnp.float32),
            pltpu.VMEM((16,), jnp.float32),
            pltpu.VMEM((16,), jnp.float32),
            pltpu.VMEM((nrows, W), jnp.float32),
            pltpu.VMEM((nrows, W), jnp.float32),
            pltpu.VMEM((pts_per_w,), jnp.float32),
            pltpu.VMEM((pts_per_w,), jnp.float32),
            pltpu.VMEM((pts_per_w,), jnp.float32),
            pltpu.VMEM((pts_per_w,), jnp.float32),
            pltpu.VMEM((pts_per_w,), jnp.float32),
            pltpu.VMEM((pts_per_w,), jnp.float32),
            pltpu.SemaphoreType.DMA,
            pltpu.SemaphoreType.DMA,
        ],
    )
    def k(bbi_hbm, depth_hbm, lv_hbm, lut_hbm,
          pts_hbm, conf_hbm,
          bboxf_v, intr_v, lut_v, rowidx_v, vprec_v, xa_v, xd_v,
          drows_v, lrows_v, p0_v, p1_v, p2_v, p3_v, p4_v, conf_v,
          semA, semB):
        wid = lax.axis_index("s") * NC + lax.axis_index("c")
        det0 = wid * dets_per_w
        b = wid // (NW // B)

        pltpu.sync_copy(bbi_hbm.at[pl.ds(det0 * 4, dets_per_w * 4)], bboxf_v)
        pltpu.sync_copy(bbi_hbm.at[pl.ds(ndet * 4, 16)], intr_v)
        pltpu.sync_copy(lut_hbm, lut_v)

        wmax = jnp.float32(W - 1)
        hmax = jnp.float32(H - 1)

        for n in range(nrows // L):
            rk4 = lut_v[pl.ds(RK4 + n * L, L)]
            rt = plsc.bitcast(lut_v[pl.ds(RT + n * L, L)], jnp.float32)
            y1 = plsc.load_gather(bboxf_v, [rk4 + 1])
            y2 = plsc.load_gather(bboxf_v, [rk4 + 3])
            y1c = jnp.clip(y1, 0.0, hmax)
            y2c = jnp.clip(y2, 0.0, hmax)
            v = y1c + rt * (y2c - y1c)
            vi = jnp.clip(v.astype(jnp.int32), 0, H - 1)
            vprec_v[pl.ds(n * L, L)] = v
            rowidx_v[pl.ds(n * L, L)] = b * H + vi

        cpA1 = pltpu.async_copy(depth_hbm.at[rowidx_v.at[pl.ds(0, half_rows)]],
                                drows_v.at[pl.ds(0, half_rows)], semA)
        cpA2 = pltpu.async_copy(lv_hbm.at[rowidx_v.at[pl.ds(0, half_rows)]],
                                lrows_v.at[pl.ds(0, half_rows)], semA)
        cpB1 = pltpu.async_copy(
            depth_hbm.at[rowidx_v.at[pl.ds(half_rows, half_rows)]],
            drows_v.at[pl.ds(half_rows, half_rows)], semB)
        cpB2 = pltpu.async_copy(
            lv_hbm.at[rowidx_v.at[pl.ds(half_rows, half_rows)]],
            lrows_v.at[pl.ds(half_rows, half_rows)], semB)

        xl = lut_v[pl.ds(XL, L)]
        x1 = plsc.load_gather(bboxf_v, [xl])
        x2 = plsc.load_gather(bboxf_v, [xl + 2])
        x1c = jnp.clip(x1, 0.0, wmax)
        x2c = jnp.clip(x2, 0.0, wmax)
        xa_v[...] = x1c
        xd_v[...] = x2c - x1c

        fxv = plsc.load_gather(intr_v, [lut_v[pl.ds(PIDX + 0 * L, L)]])
        fyv = plsc.load_gather(intr_v, [lut_v[pl.ds(PIDX + 1 * L, L)]])
        cxv = plsc.load_gather(intr_v, [lut_v[pl.ds(PIDX + 2 * L, L)]])
        cyv = plsc.load_gather(intr_v, [lut_v[pl.ds(PIDX + 3 * L, L)]])
        rfxv = jnp.float32(1.0) / fxv
        rfyv = jnp.float32(1.0) / fyv

        def point_vreg(n):
            base = n * L
            kk = lut_v[pl.ds(KK + base, L)]
            row = lut_v[pl.ds(ROW + base, L)]
            tj = plsc.bitcast(lut_v[pl.ds(TJ + base, L)], jnp.float32)
            xa = plsc.load_gather(xa_v, [kk])
            xd = plsc.load_gather(xd_v, [kk])
            u = xa + tj * xd
            v = plsc.load_gather(vprec_v, [row])
            ui = jnp.clip(u.astype(jnp.int32), 0, W - 1)
            dep = plsc.load_gather(drows_v, [row, ui])
            lv = plsc.load_gather(lrows_v, [row, ui])
            conf = jnp.clip(jnp.exp(-BETA * lv), 0.0, 1.0)
            x_cam = (u - cxv) * dep * rfxv
            y_cam = (v - cyv) * dep * rfyv
            x_r = dep
            y_r = -x_cam
            z_r = -y_cam
            mask = ((dep > 0.5)
                    & (x_r > BEV_Y_RANGE[0]) & (x_r < BEV_Y_RANGE[1])
                    & (y_r > BEV_X_RANGE[0]) & (y_r < BEV_X_RANGE[1]))
            mf = jnp.where(mask, jnp.float32(1.0), jnp.float32(0.0))
            sl = pl.ds(base, L)
            p0_v[sl] = x_r * mf
            p1_v[sl] = y_r * mf
            p2_v[sl] = z_r * mf
            p3_v[sl] = jnp.zeros((L,), jnp.float32)
            p4_v[sl] = jnp.float32(10.0) * mf
            conf_v[sl] = conf * mf

        cpA1.wait()
        cpA2.wait()
        for n in range(half_vreg):
            point_vreg(n)
        cpB1.wait()
        cpB2.wait()
        for n in range(half_vreg, nvreg):
            point_vreg(n)

        base_out = wid * pts_per_w
        pltpu.sync_copy(p0_v, pts_hbm.at[pl.ds(base_out, pts_per_w)])
        pltpu.sync_copy(p1_v, pts_hbm.at[pl.ds(npts + base_out, pts_per_w)])
        pltpu.sync_copy(p2_v, pts_hbm.at[pl.ds(2 * npts + base_out, pts_per_w)])
        pltpu.sync_copy(p3_v, pts_hbm.at[pl.ds(3 * npts + base_out, pts_per_w)])
        pltpu.sync_copy(p4_v, pts_hbm.at[pl.ds(4 * npts + base_out, pts_per_w)])
        pltpu.sync_copy(conf_v, conf_hbm.at[pl.ds(base_out, pts_per_w)])

    return k


def kernel(images, depth_map, log_var_map, bboxes, intrinsic):
    del images
    B, _, H, W = depth_map.shape
    D = bboxes.shape[1]
    ndet = B * D
    pts_per_w = (ndet // NW) * NPX * NPX
    nrows = (ndet // NW) * NPX

    depth_rows = depth_map.reshape(B * H, W)
    lv_rows = log_var_map.reshape(B * H, W)
    bbi = jnp.concatenate([
        bboxes.reshape(ndet * 4),
        intrinsic.reshape(9),
        jnp.zeros((7,), jnp.float32),
    ])

    t = np.linspace(0.0, 1.0, NPX).astype(np.float32)
    lr = np.arange(nrows)
    lp = np.arange(pts_per_w)
    ndets_w = nrows // NPX
    xlane = np.minimum(np.arange(16), ndets_w - 1) * 4
    lut = np.concatenate([
        np.repeat(np.array([0, 4, 2, 5]), 16).astype(np.int32),
        xlane.astype(np.int32),
        ((lr // NPX) * 4).astype(np.int32),
        t[lr % NPX].view(np.int32),
        (lp // (NPX * NPX)).astype(np.int32),
        ((lp // (NPX * NPX)) * NPX + (lp // NPX) % NPX).astype(np.int32),
        t[lp % NPX].view(np.int32),
    ])
    lut = jnp.asarray(lut)

    k = _make_kernel(B, D, H, W)
    pts5, conf = k(bbi, depth_rows, lv_rows, lut)
    return pts5.reshape(5, ndet * NPX * NPX).T, conf


# speedup vs baseline: 2.4389x; 1.0261x over previous
import functools

import jax
import jax.numpy as jnp
import numpy as np
from jax import lax
from jax.experimental import pallas as pl
from jax.experimental.pallas import tpu as pltpu
from jax.experimental.pallas import tpu_sc as plsc

BEV_X_RANGE = (-40.0, 40.0)
BEV_Y_RANGE = (0.0, 80.0)
BETA = 1.0
NPX = 10

NC, NS, L = 2, 16, 16
NW = NC * NS


def _make_kernel(B, D, H, W):
    ndet = B * D
    dets_per_w = ndet // NW
    pts_per_det = NPX * NPX
    pts_per_w = dets_per_w * pts_per_det
    nvreg = pts_per_w // L
    nrows = dets_per_w * NPX
    npts = ndet * pts_per_det
    half_rows = nrows // 2
    half_vreg = nvreg // 2

    mesh = plsc.VectorSubcoreMesh(core_axis_name="c", subcore_axis_name="s")

    PIDX = 0
    XL = PIDX + 64
    RK4 = XL + 16
    RT = RK4 + nrows
    KK = RT + nrows
    ROW = KK + pts_per_w
    TJ = ROW + pts_per_w
    LUT_LEN = TJ + pts_per_w

    XA = nrows

    @functools.partial(
        pl.kernel,
        out_type=(
            jax.ShapeDtypeStruct((5 * npts,), jnp.float32),
            jax.ShapeDtypeStruct((npts,), jnp.float32),
        ),
        mesh=mesh,
        compiler_params=pltpu.CompilerParams(needs_layout_passes=False),
        scratch_types=[
            pltpu.VMEM((dets_per_w * 4 + 16,), jnp.float32),
            pltpu.VMEM((LUT_LEN,), jnp.int32),
            pltpu.VMEM((nrows,), jnp.int32),
            pltpu.VMEM((nrows + 32,), jnp.float32),
            pltpu.VMEM((2 * nrows, W), jnp.float32),
            pltpu.VMEM((6 * pts_per_w,), jnp.float32),
            pltpu.SemaphoreType.DMA,
            pltpu.SemaphoreType.DMA,
        ],
    )
    def k(bbi_hbm, depth_hbm, lv_hbm, lut_hbm,
          pts_hbm, conf_hbm,
          bb_v, lut_v, rowidx_v, fprec_v, drl_v, po_v,
          semA, semB):
        wid = lax.axis_index("s") * NC + lax.axis_index("c")
        det0 = wid * dets_per_w
        b = wid // (NW // B)

        in1 = pltpu.async_copy(bbi_hbm.at[pl.ds(det0 * 4, dets_per_w * 4)],
                               bb_v.at[pl.ds(0, dets_per_w * 4)], semA)
        in2 = pltpu.async_copy(bbi_hbm.at[pl.ds(ndet * 4, 16)],
                               bb_v.at[pl.ds(dets_per_w * 4, 16)], semA)
        in3 = pltpu.async_copy(lut_hbm, lut_v, semA)
        in1.wait()
        in2.wait()
        in3.wait()

        wmax = jnp.float32(W - 1)
        hmax = jnp.float32(H - 1)

        for n in range(nrows // L):
            rk4 = lut_v[pl.ds(RK4 + n * L, L)]
            rt = plsc.bitcast(lut_v[pl.ds(RT + n * L, L)], jnp.float32)
            y1 = plsc.load_gather(bb_v, [rk4 + 1])
            y2 = plsc.load_gather(bb_v, [rk4 + 3])
            y1c = jnp.clip(y1, 0.0, hmax)
            y2c = jnp.clip(y2, 0.0, hmax)
            v = y1c + rt * (y2c - y1c)
            vi = jnp.clip(v.astype(jnp.int32), 0, H - 1)
            fprec_v[pl.ds(n * L, L)] = v
            rowidx_v[pl.ds(n * L, L)] = b * H + vi

        cpA1 = pltpu.async_copy(depth_hbm.at[rowidx_v.at[pl.ds(0, half_rows)]],
                                drl_v.at[pl.ds(0, half_rows)], semA)
        cpA2 = pltpu.async_copy(lv_hbm.at[rowidx_v.at[pl.ds(0, half_rows)]],
                                drl_v.at[pl.ds(nrows, half_rows)], semA)
        cpB1 = pltpu.async_copy(
            depth_hbm.at[rowidx_v.at[pl.ds(half_rows, half_rows)]],
            drl_v.at[pl.ds(half_rows, half_rows)], semB)
        cpB2 = pltpu.async_copy(
            lv_hbm.at[rowidx_v.at[pl.ds(half_rows, half_rows)]],
            drl_v.at[pl.ds(nrows + half_rows, half_rows)], semB)

        xl = lut_v[pl.ds(XL, L)]
        x1 = plsc.load_gather(bb_v, [xl])
        x2 = plsc.load_gather(bb_v, [xl + 2])
        x1c = jnp.clip(x1, 0.0, wmax)
        x2c = jnp.clip(x2, 0.0, wmax)
        fprec_v[pl.ds(XA, L)] = x1c
        fprec_v[pl.ds(XA + L, L)] = x2c - x1c

        fxv = plsc.load_gather(bb_v, [lut_v[pl.ds(PIDX + 0 * L, L)]])
        fyv = plsc.load_gather(bb_v, [lut_v[pl.ds(PIDX + 1 * L, L)]])
        cxv = plsc.load_gather(bb_v, [lut_v[pl.ds(PIDX + 2 * L, L)]])
        cyv = plsc.load_gather(bb_v, [lut_v[pl.ds(PIDX + 3 * L, L)]])
        rfxv = jnp.float32(1.0) / fxv
        rfyv = jnp.float32(1.0) / fyv

        def point_vreg(n):
            base = n * L
            kk = lut_v[pl.ds(KK + base, L)]
            row = lut_v[pl.ds(ROW + base, L)]
            tj = plsc.bitcast(lut_v[pl.ds(TJ + base, L)], jnp.float32)
            xa = plsc.load_gather(fprec_v, [kk])
            xd = plsc.load_gather(fprec_v, [kk + L])
            u = xa + tj * xd
            v = plsc.load_gather(fprec_v, [row])
            ui = jnp.clip(u.astype(jnp.int32), 0, W - 1)
            dep = plsc.load_gather(drl_v, [row, ui])
            lv = plsc.load_gather(drl_v, [row + nrows, ui])
            conf = jnp.clip(jnp.exp(-BETA * lv), 0.0, 1.0)
            x_cam = (u - cxv) * dep * rfxv
            y_cam = (v - cyv) * dep * rfyv
            x_r = dep
            y_r = -x_cam
            z_r = -y_cam
            mask = ((dep > 0.5)
                    & (x_r > BEV_Y_RANGE[0]) & (x_r < BEV_Y_RANGE[1])
                    & (y_r > BEV_X_RANGE[0]) & (y_r < BEV_X_RANGE[1]))
            mf = jnp.where(mask, jnp.float32(1.0), jnp.float32(0.0))
            po_v[pl.ds(base, L)] = x_r * mf
            po_v[pl.ds(pts_per_w + base, L)] = y_r * mf
            po_v[pl.ds(2 * pts_per_w + base, L)] = z_r * mf
            po_v[pl.ds(3 * pts_per_w + base, L)] = jnp.zeros((L,), jnp.float32)
            po_v[pl.ds(4 * pts_per_w + base, L)] = jnp.float32(10.0) * mf
            po_v[pl.ds(5 * pts_per_w + base, L)] = conf * mf

        cpA1.wait()
        cpA2.wait()
        for n in range(half_vreg):
            point_vreg(n)
        cpB1.wait()
        cpB2.wait()
        for n in range(half_vreg, nvreg):
            point_vreg(n)

        base_out = wid * pts_per_w
        outs = []
        for c in range(5):
            outs.append(pltpu.async_copy(
                po_v.at[pl.ds(c * pts_per_w, pts_per_w)],
                pts_hbm.at[pl.ds(c * npts + base_out, pts_per_w)], semB))
        outs.append(pltpu.async_copy(
            po_v.at[pl.ds(5 * pts_per_w, pts_per_w)],
            conf_hbm.at[pl.ds(base_out, pts_per_w)], semB))
        for cp in outs:
            cp.wait()

    return k


def kernel(images, depth_map, log_var_map, bboxes, intrinsic):
    del images
    B, _, H, W = depth_map.shape
    D = bboxes.shape[1]
    ndet = B * D
    pts_per_w = (ndet // NW) * NPX * NPX
    nrows = (ndet // NW) * NPX
    ndets_w = nrows // NPX

    depth_rows = depth_map.reshape(B * H, W)
    lv_rows = log_var_map.reshape(B * H, W)
    bbi = jnp.concatenate([
        bboxes.reshape(ndet * 4),
        intrinsic.reshape(9),
        jnp.zeros((7,), jnp.float32),
    ])

    t = np.linspace(0.0, 1.0, NPX).astype(np.float32)
    lr = np.arange(nrows)
    lp = np.arange(pts_per_w)
    xlane = np.minimum(np.arange(16), ndets_w - 1) * 4
    ioff = ndet // NW * 4
    lut = np.concatenate([
        (np.repeat(np.array([0, 4, 2, 5]), 16) + ioff).astype(np.int32),
        xlane.astype(np.int32),
        ((lr // NPX) * 4).astype(np.int32),
        t[lr % NPX].view(np.int32),
        ((lp // (NPX * NPX)) + nrows).astype(np.int32),
        ((lp // (NPX * NPX)) * NPX + (lp // NPX) % NPX).astype(np.int32),
        t[lp % NPX].view(np.int32),
    ])
    lut = jnp.asarray(lut)

    k = _make_kernel(B, D, H, W)
    pts5, conf = k(bbi, depth_rows, lv_rows, lut)
    return pts5.reshape(5, ndet * NPX * NPX).T, conf


# speedup vs baseline: 2.6822x; 1.0998x over previous
import functools

import jax
import jax.numpy as jnp
import numpy as np
from jax import lax
from jax.experimental import pallas as pl
from jax.experimental.pallas import tpu as pltpu
from jax.experimental.pallas import tpu_sc as plsc

BEV_X_RANGE = (-40.0, 40.0)
BEV_Y_RANGE = (0.0, 80.0)
BETA = 1.0
NPX = 10

NC, NS, L = 2, 16, 16
NW = NC * NS


def _make_kernel(B, D, H, W):
    ndet = B * D
    dets_per_w = ndet // NW
    pts_per_det = NPX * NPX
    pts_per_w = dets_per_w * pts_per_det
    nvreg = pts_per_w // L
    nrows = dets_per_w * NPX
    npts = ndet * pts_per_det
    half_rows = nrows // 2
    half_vreg = nvreg // 2

    mesh = plsc.VectorSubcoreMesh(core_axis_name="c", subcore_axis_name="s")

    PIDX = 0
    XL = PIDX + 64
    RK4 = XL + 16
    RT = RK4 + nrows
    KK = RT + nrows
    ROW = KK + pts_per_w
    TJ = ROW + pts_per_w
    LUT_LEN = TJ + pts_per_w

    XA = nrows

    @functools.partial(
        pl.kernel,
        out_type=(
            jax.ShapeDtypeStruct((5 * npts,), jnp.float32),
            jax.ShapeDtypeStruct((npts,), jnp.float32),
        ),
        mesh=mesh,
        compiler_params=pltpu.CompilerParams(needs_layout_passes=False),
        scratch_types=[
            pltpu.VMEM((dets_per_w * 4 + 16,), jnp.float32),
            pltpu.VMEM((LUT_LEN,), jnp.int32),
            pltpu.VMEM((nrows,), jnp.int32),
            pltpu.VMEM((nrows + 32,), jnp.float32),
            pltpu.VMEM((2 * nrows, W), jnp.float32),
            pltpu.VMEM((6 * pts_per_w,), jnp.float32),
            pltpu.SemaphoreType.DMA,
            pltpu.SemaphoreType.DMA,
        ],
    )
    def k(bbi_hbm, depth_hbm, lv_hbm, lut_hbm,
          pts_hbm, conf_hbm,
          bb_v, lut_v, rowidx_v, fprec_v, drl_v, po_v,
          semA, semB):
        wid = lax.axis_index("s") * NC + lax.axis_index("c")
        det0 = wid * dets_per_w
        b = wid // (NW // B)

        in1 = pltpu.async_copy(bbi_hbm.at[pl.ds(det0 * 4, dets_per_w * 4)],
                               bb_v.at[pl.ds(0, dets_per_w * 4)], semA)
        in2 = pltpu.async_copy(bbi_hbm.at[pl.ds(ndet * 4, 16)],
                               bb_v.at[pl.ds(dets_per_w * 4, 16)], semA)
        in3 = pltpu.async_copy(lut_hbm, lut_v, semA)
        in1.wait()
        in2.wait()
        in3.wait()

        wmax = jnp.float32(W - 1)
        hmax = jnp.float32(H - 1)

        for n in range(nrows // L):
            rk4 = lut_v[pl.ds(RK4 + n * L, L)]
            rt = plsc.bitcast(lut_v[pl.ds(RT + n * L, L)], jnp.float32)
            y1 = plsc.load_gather(bb_v, [rk4 + 1])
            y2 = plsc.load_gather(bb_v, [rk4 + 3])
            y1c = jnp.clip(y1, 0.0, hmax)
            y2c = jnp.clip(y2, 0.0, hmax)
            v = y1c + rt * (y2c - y1c)
            vi = jnp.clip(v.astype(jnp.int32), 0, H - 1)
            fprec_v[pl.ds(n * L, L)] = v
            rowidx_v[pl.ds(n * L, L)] = b * H + vi

        cpA1 = pltpu.async_copy(depth_hbm.at[rowidx_v.at[pl.ds(0, half_rows)]],
                                drl_v.at[pl.ds(0, half_rows)], semA)
        cpA2 = pltpu.async_copy(lv_hbm.at[rowidx_v.at[pl.ds(0, half_rows)]],
                                drl_v.at[pl.ds(nrows, half_rows)], semA)
        cpB1 = pltpu.async_copy(
            depth_hbm.at[rowidx_v.at[pl.ds(half_rows, half_rows)]],
            drl_v.at[pl.ds(half_rows, half_rows)], semB)
        cpB2 = pltpu.async_copy(
            lv_hbm.at[rowidx_v.at[pl.ds(half_rows, half_rows)]],
            drl_v.at[pl.ds(nrows + half_rows, half_rows)], semB)

        xl = lut_v[pl.ds(XL, L)]
        x1 = plsc.load_gather(bb_v, [xl])
        x2 = plsc.load_gather(bb_v, [xl + 2])
        x1c = jnp.clip(x1, 0.0, wmax)
        x2c = jnp.clip(x2, 0.0, wmax)
        fprec_v[pl.ds(XA, L)] = x1c
        fprec_v[pl.ds(XA + L, L)] = x2c - x1c

        fxv = plsc.load_gather(bb_v, [lut_v[pl.ds(PIDX + 0 * L, L)]])
        fyv = plsc.load_gather(bb_v, [lut_v[pl.ds(PIDX + 1 * L, L)]])
        cxv = plsc.load_gather(bb_v, [lut_v[pl.ds(PIDX + 2 * L, L)]])
        cyv = plsc.load_gather(bb_v, [lut_v[pl.ds(PIDX + 3 * L, L)]])
        rfxv = jnp.float32(1.0) / fxv
        rfyv = jnp.float32(1.0) / fyv

        def point_vreg(n):
            base = n * L
            kk = lut_v[pl.ds(KK + base, L)]
            row = lut_v[pl.ds(ROW + base, L)]
            tj = plsc.bitcast(lut_v[pl.ds(TJ + base, L)], jnp.float32)
            xa = plsc.load_gather(fprec_v, [kk])
            xd = plsc.load_gather(fprec_v, [kk + L])
            u = xa + tj * xd
            v = plsc.load_gather(fprec_v, [row])
            ui = jnp.clip(u.astype(jnp.int32), 0, W - 1)
            dep = plsc.load_gather(drl_v, [row, ui])
            lv = plsc.load_gather(drl_v, [row + nrows, ui])
            conf = jnp.clip(jnp.exp(-BETA * lv), 0.0, 1.0)
            x_cam = (u - cxv) * dep * rfxv
            y_cam = (v - cyv) * dep * rfyv
            x_r = dep
            y_r = -x_cam
            z_r = -y_cam
            mask = ((dep > 0.5)
                    & (x_r > BEV_Y_RANGE[0]) & (x_r < BEV_Y_RANGE[1])
                    & (y_r > BEV_X_RANGE[0]) & (y_r < BEV_X_RANGE[1]))
            mf = jnp.where(mask, jnp.float32(1.0), jnp.float32(0.0))
            po_v[pl.ds(base, L)] = x_r * mf
            po_v[pl.ds(pts_per_w + base, L)] = y_r * mf
            po_v[pl.ds(2 * pts_per_w + base, L)] = z_r * mf
            po_v[pl.ds(3 * pts_per_w + base, L)] = jnp.zeros((L,), jnp.float32)
            po_v[pl.ds(4 * pts_per_w + base, L)] = jnp.float32(10.0) * mf
            po_v[pl.ds(5 * pts_per_w + base, L)] = conf * mf

        def body(n, carry):
            point_vreg(n)
            return carry

        cpA1.wait()
        cpA2.wait()
        lax.fori_loop(0, half_vreg, body, 0)
        cpB1.wait()
        cpB2.wait()
        lax.fori_loop(half_vreg, nvreg, body, 0)

        base_out = wid * pts_per_w
        outs = []
        for c in range(5):
            outs.append(pltpu.async_copy(
                po_v.at[pl.ds(c * pts_per_w, pts_per_w)],
                pts_hbm.at[pl.ds(c * npts + base_out, pts_per_w)], semB))
        outs.append(pltpu.async_copy(
            po_v.at[pl.ds(5 * pts_per_w, pts_per_w)],
            conf_hbm.at[pl.ds(base_out, pts_per_w)], semB))
        for cp in outs:
            cp.wait()

    return k


def kernel(images, depth_map, log_var_map, bboxes, intrinsic):
    del images
    B, _, H, W = depth_map.shape
    D = bboxes.shape[1]
    ndet = B * D
    pts_per_w = (ndet // NW) * NPX * NPX
    nrows = (ndet // NW) * NPX
    ndets_w = nrows // NPX

    depth_rows = depth_map.reshape(B * H, W)
    lv_rows = log_var_map.reshape(B * H, W)
    bbi = jnp.concatenate([
        bboxes.reshape(ndet * 4),
        intrinsic.reshape(9),
        jnp.zeros((7,), jnp.float32),
    ])

    t = np.linspace(0.0, 1.0, NPX).astype(np.float32)
    lr = np.arange(nrows)
    lp = np.arange(pts_per_w)
    xlane = np.minimum(np.arange(16), ndets_w - 1) * 4
    ioff = ndet // NW * 4
    lut = np.concatenate([
        (np.repeat(np.array([0, 4, 2, 5]), 16) + ioff).astype(np.int32),
        xlane.astype(np.int32),
        ((lr // NPX) * 4).astype(np.int32),
        t[lr % NPX].view(np.int32),
        ((lp // (NPX * NPX)) + nrows).astype(np.int32),
        ((lp // (NPX * NPX)) * NPX + (lp // NPX) % NPX).astype(np.int32),
        t[lp % NPX].view(np.int32),
    ])
    lut = jnp.asarray(lut)

    k = _make_kernel(B, D, H, W)
    pts5, conf = k(bbi, depth_rows, lv_rows, lut)
    return pts5.reshape(5, ndet * NPX * NPX).T, conf


# speedup vs baseline: 2.6947x; 1.0047x over previous
import functools

import jax
import jax.numpy as jnp
import numpy as np
from jax import lax
from jax.experimental import pallas as pl
from jax.experimental.pallas import tpu as pltpu
from jax.experimental.pallas import tpu_sc as plsc

BEV_X_RANGE = (-40.0, 40.0)
BEV_Y_RANGE = (0.0, 80.0)
BETA = 1.0
NPX = 10

NC, NS, L = 2, 16, 16
NW = NC * NS


def _make_kernel(B, D, H, W):
    ndet = B * D
    dets_per_w = ndet // NW
    pts_per_det = NPX * NPX
    pts_per_w = dets_per_w * pts_per_det
    nvreg = pts_per_w // L
    nrows = dets_per_w * NPX
    npts = ndet * pts_per_det
    half_rows = nrows // 2
    half_vreg = nvreg // 2

    mesh = plsc.VectorSubcoreMesh(core_axis_name="c", subcore_axis_name="s")

    PIDX = 0
    XL = PIDX + 64
    RK4 = XL + 16
    RT = RK4 + nrows
    KK = RT + nrows
    ROW = KK + pts_per_w
    TJ = ROW + pts_per_w
    LUT_LEN = TJ + pts_per_w

    XA = nrows

    @functools.partial(
        pl.kernel,
        out_type=(
            jax.ShapeDtypeStruct((5 * npts,), jnp.float32),
            jax.ShapeDtypeStruct((npts,), jnp.float32),
        ),
        mesh=mesh,
        compiler_params=pltpu.CompilerParams(needs_layout_passes=False),
        scratch_types=[
            pltpu.VMEM((dets_per_w * 4 + 16,), jnp.float32),
            pltpu.VMEM((LUT_LEN,), jnp.int32),
            pltpu.VMEM((nrows,), jnp.int32),
            pltpu.VMEM((nrows + 32,), jnp.float32),
            pltpu.VMEM((2 * nrows, W), jnp.float32),
            pltpu.VMEM((6 * pts_per_w,), jnp.float32),
            pltpu.SemaphoreType.DMA,
            pltpu.SemaphoreType.DMA,
        ],
    )
    def k(bbi_hbm, depth_hbm, lv_hbm, lut_hbm,
          pts_hbm, conf_hbm,
          bb_v, lut_v, rowidx_v, fprec_v, drl_v, po_v,
          semA, semB):
        wid = lax.axis_index("s") * NC + lax.axis_index("c")
        det0 = wid * dets_per_w
        b = wid // (NW // B)

        in1 = pltpu.async_copy(bbi_hbm.at[pl.ds(det0 * 4, dets_per_w * 4)],
                               bb_v.at[pl.ds(0, dets_per_w * 4)], semA)
        in2 = pltpu.async_copy(bbi_hbm.at[pl.ds(ndet * 4, 16)],
                               bb_v.at[pl.ds(dets_per_w * 4, 16)], semA)
        in3 = pltpu.async_copy(lut_hbm, lut_v, semA)
        in1.wait()
        in2.wait()
        in3.wait()

        wmax = jnp.float32(W - 1)
        hmax = jnp.float32(H - 1)

        def stage1(n, carry):
            rk4 = lut_v[pl.ds(RK4 + n * L, L)]
            rt = plsc.bitcast(lut_v[pl.ds(RT + n * L, L)], jnp.float32)
            y1 = plsc.load_gather(bb_v, [rk4 + 1])
            y2 = plsc.load_gather(bb_v, [rk4 + 3])
            y1c = jnp.clip(y1, 0.0, hmax)
            y2c = jnp.clip(y2, 0.0, hmax)
            v = y1c + rt * (y2c - y1c)
            vi = jnp.clip(v.astype(jnp.int32), 0, H - 1)
            fprec_v[pl.ds(n * L, L)] = v
            rowidx_v[pl.ds(n * L, L)] = b * H + vi
            return carry

        lax.fori_loop(0, nrows // L, stage1, 0)

        cpA1 = pltpu.async_copy(depth_hbm.at[rowidx_v.at[pl.ds(0, half_rows)]],
                                drl_v.at[pl.ds(0, half_rows)], semA)
        cpA2 = pltpu.async_copy(lv_hbm.at[rowidx_v.at[pl.ds(0, half_rows)]],
                                drl_v.at[pl.ds(nrows, half_rows)], semA)
        cpB1 = pltpu.async_copy(
            depth_hbm.at[rowidx_v.at[pl.ds(half_rows, half_rows)]],
            drl_v.at[pl.ds(half_rows, half_rows)], semB)
        cpB2 = pltpu.async_copy(
            lv_hbm.at[rowidx_v.at[pl.ds(half_rows, half_rows)]],
            drl_v.at[pl.ds(nrows + half_rows, half_rows)], semB)

        xl = lut_v[pl.ds(XL, L)]
        x1 = plsc.load_gather(bb_v, [xl])
        x2 = plsc.load_gather(bb_v, [xl + 2])
        x1c = jnp.clip(x1, 0.0, wmax)
        x2c = jnp.clip(x2, 0.0, wmax)
        fprec_v[pl.ds(XA, L)] = x1c
        fprec_v[pl.ds(XA + L, L)] = x2c - x1c

        fxv = plsc.load_gather(bb_v, [lut_v[pl.ds(PIDX + 0 * L, L)]])
        fyv = plsc.load_gather(bb_v, [lut_v[pl.ds(PIDX + 1 * L, L)]])
        cxv = plsc.load_gather(bb_v, [lut_v[pl.ds(PIDX + 2 * L, L)]])
        cyv = plsc.load_gather(bb_v, [lut_v[pl.ds(PIDX + 3 * L, L)]])
        rfxv = jnp.float32(1.0) / fxv
        rfyv = jnp.float32(1.0) / fyv

        def point_vreg(n):
            base = n * L
            kk = lut_v[pl.ds(KK + base, L)]
            row = lut_v[pl.ds(ROW + base, L)]
            tj = plsc.bitcast(lut_v[pl.ds(TJ + base, L)], jnp.float32)
            xa = plsc.load_gather(fprec_v, [kk])
            xd = plsc.load_gather(fprec_v, [kk + L])
            u = xa + tj * xd
            v = plsc.load_gather(fprec_v, [row])
            ui = jnp.clip(u.astype(jnp.int32), 0, W - 1)
            dep = plsc.load_gather(drl_v, [row, ui])
            lv = plsc.load_gather(drl_v, [row + nrows, ui])
            conf = jnp.clip(jnp.exp(-BETA * lv), 0.0, 1.0)
            x_cam = (u - cxv) * dep * rfxv
            y_cam = (v - cyv) * dep * rfyv
            x_r = dep
            y_r = -x_cam
            z_r = -y_cam
            mask = ((dep > 0.5)
                    & (x_r > BEV_Y_RANGE[0]) & (x_r < BEV_Y_RANGE[1])
                    & (y_r > BEV_X_RANGE[0]) & (y_r < BEV_X_RANGE[1]))
            mf = jnp.where(mask, jnp.float32(1.0), jnp.float32(0.0))
            po_v[pl.ds(base, L)] = x_r * mf
            po_v[pl.ds(pts_per_w + base, L)] = y_r * mf
            po_v[pl.ds(2 * pts_per_w + base, L)] = z_r * mf
            po_v[pl.ds(3 * pts_per_w + base, L)] = jnp.zeros((L,), jnp.float32)
            po_v[pl.ds(4 * pts_per_w + base, L)] = jnp.float32(10.0) * mf
            po_v[pl.ds(5 * pts_per_w + base, L)] = conf * mf

        def body(n, carry):
            point_vreg(n)
            return carry

        cpA1.wait()
        cpA2.wait()
        lax.fori_loop(0, half_vreg, body, 0)
        cpB1.wait()
        cpB2.wait()
        lax.fori_loop(half_vreg, nvreg, body, 0)

        base_out = wid * pts_per_w
        outs = []
        for c in range(5):
            outs.append(pltpu.async_copy(
                po_v.at[pl.ds(c * pts_per_w, pts_per_w)],
                pts_hbm.at[pl.ds(c * npts + base_out, pts_per_w)], semB))
        outs.append(pltpu.async_copy(
            po_v.at[pl.ds(5 * pts_per_w, pts_per_w)],
            conf_hbm.at[pl.ds(base_out, pts_per_w)], semB))
        for cp in outs:
            cp.wait()

    return k


def kernel(images, depth_map, log_var_map, bboxes, intrinsic):
    del images
    B, _, H, W = depth_map.shape
    D = bboxes.shape[1]
    ndet = B * D
    pts_per_w = (ndet // NW) * NPX * NPX
    nrows = (ndet // NW) * NPX
    ndets_w = nrows // NPX

    depth_rows = depth_map.reshape(B * H, W)
    lv_rows = log_var_map.reshape(B * H, W)
    bbi = jnp.concatenate([
        bboxes.reshape(ndet * 4),
        intrinsic.reshape(9),
        jnp.zeros((7,), jnp.float32),
    ])

    t = np.linspace(0.0, 1.0, NPX).astype(np.float32)
    lr = np.arange(nrows)
    lp = np.arange(pts_per_w)
    xlane = np.minimum(np.arange(16), ndets_w - 1) * 4
    ioff = ndet // NW * 4
    lut = np.concatenate([
        (np.repeat(np.array([0, 4, 2, 5]), 16) + ioff).astype(np.int32),
        xlane.astype(np.int32),
        ((lr // NPX) * 4).astype(np.int32),
        t[lr % NPX].view(np.int32),
        ((lp // (NPX * NPX)) + nrows).astype(np.int32),
        ((lp // (NPX * NPX)) * NPX + (lp // NPX) % NPX).astype(np.int32),
        t[lp % NPX].view(np.int32),
    ])
    lut = jnp.asarray(lut)

    k = _make_kernel(B, D, H, W)
    pts5, conf = k(bbi, depth_rows, lv_rows, lut)
    return pts5.reshape(5, ndet * NPX * NPX).T, conf
